# baseline (speedup 1.0000x reference)
import functools

import jax
import jax.numpy as jnp
from jax import lax
from jax.experimental import pallas as pl
from jax.experimental.pallas import tpu as pltpu
from jax.experimental.pallas import tpu_sc as plsc

N_SEEDS = 1_000_000
PLANE = 2097152
CHUNK = 2000
W_CHUNK = 3 * CHUNK
N_CHUNKS = N_SEEDS // CHUNK
NC, NS = 2, 16
NW = NC * NS
GROUPS = CHUNK // 16

_mesh = plsc.VectorSubcoreMesh(core_axis_name="c", subcore_axis_name="s",
                               num_cores=NC, num_subcores=NS)


@functools.partial(
    pl.kernel,
    out_type=jax.ShapeDtypeStruct((3 * N_SEEDS,), jnp.float32),
    mesh=_mesh,
    scratch_types=[
        pltpu.VMEM((W_CHUNK,), jnp.int32),
        pltpu.VMEM((W_CHUNK,), jnp.int32),
        pltpu.VMEM((W_CHUNK,), jnp.float32),
        pltpu.SemaphoreType.DMA,
    ],
    compiler_params=pltpu.CompilerParams(needs_layout_passes=False),
)
def _gather(seeds_hbm, table_hbm, out_hbm, seeds_v, idx_v, dest_v, sem):
    wid = lax.axis_index("s") * NC + lax.axis_index("c")
    offs3 = lax.iota(jnp.int32, 16) * 3

    def chunk_body(t, _):
        k = wid + t * NW

        @pl.when(k < N_CHUNKS)
        def _():
            base_w = k * W_CHUNK
            pltpu.sync_copy(seeds_hbm.at[pl.ds(base_w, W_CHUNK)], seeds_v)

            def group_body(g, _):
                b = g * 48
                x = plsc.load_gather(seeds_v, [b + offs3])
                y = plsc.load_gather(seeds_v, [b + offs3 + 1])
                z = plsc.load_gather(seeds_v, [b + offs3 + 2])
                flat = (x << 14) | (y << 7) | z
                plsc.store_scatter(idx_v, [b + offs3], flat)
                plsc.store_scatter(idx_v, [b + offs3 + 1], flat + PLANE)
                plsc.store_scatter(idx_v, [b + offs3 + 2], flat + 2 * PLANE)
                return 0

            lax.fori_loop(0, GROUPS, group_body, 0)
            pltpu.async_copy(table_hbm.at[idx_v], dest_v, sem).wait()
            pltpu.sync_copy(dest_v, out_hbm.at[pl.ds(base_w, W_CHUNK)])

        return 0

    lax.fori_loop(0, (N_CHUNKS + NW - 1) // NW, chunk_body, 0)


def kernel(seeds, vector_field):
    seeds_flat = seeds.reshape(3 * N_SEEDS)
    table = vector_field.reshape(3 * PLANE)
    out = _gather(seeds_flat, table)
    return out.reshape(N_SEEDS, 3)


# speedup vs baseline: 17.5522x; 17.5522x over previous
import functools

import jax
import jax.numpy as jnp
from jax import lax
from jax.experimental import pallas as pl
from jax.experimental.pallas import tpu as pltpu
from jax.experimental.pallas import tpu_sc as plsc

N_SEEDS = 1_000_000
PLANE = 2097152
CHUNK = 2000
W_CHUNK = 3 * CHUNK
N_CHUNKS = N_SEEDS // CHUNK
NC, NS = 2, 16
NW = NC * NS
GROUPS = CHUNK // 16

_mesh = plsc.VectorSubcoreMesh(core_axis_name="c", subcore_axis_name="s",
                               num_cores=NC, num_subcores=NS)


@functools.partial(
    pl.kernel,
    out_type=jax.ShapeDtypeStruct((3 * N_SEEDS,), jnp.float32),
    mesh=_mesh,
    scratch_types=[
        pltpu.VMEM((W_CHUNK,), jnp.int32),
        pltpu.VMEM((W_CHUNK,), jnp.int32),
        pltpu.VMEM((W_CHUNK,), jnp.float32),
        pltpu.SemaphoreType.DMA,
    ],
    compiler_params=pltpu.CompilerParams(needs_layout_passes=False),
)
def _gather(seeds_hbm, table_hbm, out_hbm, sv, idx_v, dest_v, sem):
    wid = lax.axis_index("s") * NC + lax.axis_index("c")

    def chunk_body(t, _):
        k = wid + t * NW

        @pl.when(k < N_CHUNKS)
        def _():
            base = k * CHUNK
            for c in range(3):
                pltpu.sync_copy(seeds_hbm.at[pl.ds(c * N_SEEDS + base, CHUNK)],
                                sv.at[pl.ds(c * CHUNK, CHUNK)])

            def group_body(g, _):
                s = g * 16
                x = sv[pl.ds(s, 16)]
                y = sv[pl.ds(CHUNK + s, 16)]
                z = sv[pl.ds(2 * CHUNK + s, 16)]
                flat = (x << 14) | (y << 7) | z
                idx_v[pl.ds(s, 16)] = flat
                idx_v[pl.ds(CHUNK + s, 16)] = flat + PLANE
                idx_v[pl.ds(2 * CHUNK + s, 16)] = flat + 2 * PLANE
                return 0

            lax.fori_loop(0, GROUPS, group_body, 0)
            pltpu.async_copy(table_hbm.at[idx_v], dest_v, sem).wait()
            for c in range(3):
                pltpu.sync_copy(dest_v.at[pl.ds(c * CHUNK, CHUNK)],
                                out_hbm.at[pl.ds(c * N_SEEDS + base, CHUNK)])

        return 0

    lax.fori_loop(0, (N_CHUNKS + NW - 1) // NW, chunk_body, 0)


def kernel(seeds, vector_field):
    seeds_planar = seeds.T.reshape(3 * N_SEEDS)
    table = vector_field.reshape(3 * PLANE)
    out = _gather(seeds_planar, table)
    return out.reshape(3, N_SEEDS).T


# speedup vs baseline: 21.1379x; 1.2043x over previous
import functools

import jax
import jax.numpy as jnp
from jax import lax
from jax.experimental import pallas as pl
from jax.experimental.pallas import tpu as pltpu
from jax.experimental.pallas import tpu_sc as plsc

N_SEEDS = 1_000_000
PLANE = 2097152
CHUNK = 2000
W_CHUNK = 3 * CHUNK
N_CHUNKS = N_SEEDS // CHUNK
NC, NS = 2, 16
NW = NC * NS
GROUPS = CHUNK // 16
T_STEPS = (N_CHUNKS + NW - 1) // NW

_mesh = plsc.VectorSubcoreMesh(core_axis_name="c", subcore_axis_name="s",
                               num_cores=NC, num_subcores=NS)


@functools.partial(
    pl.kernel,
    out_type=jax.ShapeDtypeStruct((3 * N_SEEDS,), jnp.float32),
    mesh=_mesh,
    scratch_types=[
        pltpu.VMEM((W_CHUNK,), jnp.int32),
        pltpu.VMEM((W_CHUNK,), jnp.int32),
        pltpu.VMEM((W_CHUNK,), jnp.int32),
        pltpu.VMEM((W_CHUNK,), jnp.int32),
        pltpu.VMEM((W_CHUNK,), jnp.float32),
        pltpu.VMEM((W_CHUNK,), jnp.float32),
        pltpu.SemaphoreType.DMA,
        pltpu.SemaphoreType.DMA,
        pltpu.SemaphoreType.DMA,
    ],
    compiler_params=pltpu.CompilerParams(needs_layout_passes=False),
)
def _gather(seeds_hbm, table_hbm, out_hbm,
            sv_a, sv_b, idx_a, idx_b, dest_a, dest_b, sem_g, sem_oa, sem_ob):
    wid = lax.axis_index("s") * NC + lax.axis_index("c")
    bufs = [(sv_a, idx_a, dest_a, sem_oa), (sv_b, idx_b, dest_b, sem_ob)]

    def load_and_index(k, sv, idxv):
        base = k * CHUNK
        for c in range(3):
            pltpu.sync_copy(seeds_hbm.at[pl.ds(c * N_SEEDS + base, CHUNK)],
                            sv.at[pl.ds(c * CHUNK, CHUNK)])

        def group_body(g, _):
            s = g * 16
            x = sv[pl.ds(s, 16)]
            y = sv[pl.ds(CHUNK + s, 16)]
            z = sv[pl.ds(2 * CHUNK + s, 16)]
            flat = (x << 14) | (y << 7) | z
            idxv[pl.ds(s, 16)] = flat
            idxv[pl.ds(CHUNK + s, 16)] = flat + PLANE
            idxv[pl.ds(2 * CHUNK + s, 16)] = flat + 2 * PLANE
            return 0

        lax.fori_loop(0, GROUPS, group_body, 0)

    def issue_outs(k, dest, sem):
        base = k * CHUNK
        for c in range(3):
            pltpu.async_copy(dest.at[pl.ds(c * CHUNK, CHUNK)],
                             out_hbm.at[pl.ds(c * N_SEEDS + base, CHUNK)], sem)

    def drain_outs(dest, sem):
        for c in range(3):
            pltpu.make_async_copy(dest.at[pl.ds(c * CHUNK, CHUNK)],
                                  out_hbm.at[pl.ds(0, CHUNK)], sem).wait()

    pending_g = [None, None]

    for t in range(T_STEPS):
        b = t & 1
        sv, idxv, dest, sem_o = bufs[b]
        p_sv, p_idxv, p_dest, p_sem_o = bufs[1 - b]
        k = wid + t * NW

        @pl.when(k < N_CHUNKS)
        def _(t=t, b=b, k=k, sv=sv, idxv=idxv, dest=dest, sem_o=sem_o,
              p_dest=p_dest, p_sem_o=p_sem_o):
            if t >= 2:
                drain_outs(dest, sem_o)
            load_and_index(k, sv, idxv)
            if t >= 1:
                pending_g[1 - b].wait()
                issue_outs(k - NW, p_dest, p_sem_o)
            h = pltpu.async_copy(table_hbm.at[idxv], dest, sem_g)
            pending_g[b] = h

    last_k = wid + (T_STEPS - 1) * NW
    lb = (T_STEPS - 1) & 1

    @pl.when(last_k < N_CHUNKS)
    def _():
        pending_g[lb].wait()
        issue_outs(last_k, bufs[lb][2], bufs[lb][3])
        drain_outs(bufs[1 - lb][2], bufs[1 - lb][3])
        drain_outs(bufs[lb][2], bufs[lb][3])

    @pl.when(last_k >= N_CHUNKS)
    def _():
        pending_g[1 - lb].wait()
        issue_outs(last_k - NW, bufs[1 - lb][2], bufs[1 - lb][3])
        drain_outs(bufs[lb][2], bufs[lb][3])
        drain_outs(bufs[1 - lb][2], bufs[1 - lb][3])


def kernel(seeds, vector_field):
    seeds_planar = seeds.T.reshape(3 * N_SEEDS)
    table = vector_field.reshape(3 * PLANE)
    out = _gather(seeds_planar, table)
    return out.reshape(3, N_SEEDS).T
